# Initial kernel scaffold; baseline (speedup 1.0000x reference)
import jax
import jax.numpy as jnp
from jax.experimental import pallas as pl


def kernel(xy, W1, b1, W2, b2):
    raise NotImplementedError("write your pallas kernel here")


# speedup vs baseline: 16.1104x; 16.1104x over previous
import functools

import jax
import jax.numpy as jnp
from jax import lax
from jax.experimental import pallas as pl
from jax.experimental.pallas import tpu as pltpu
from jax.experimental.pallas import tpu_sc as plsc

BS = 8
PD = 2
N = 2048
K = 10
EMB = 128

RB = 512
NBLK = N // RB
NSLAB = BS * NBLK

NC = 2
NS = 16
NW = NC * NS
ROWS_PER_W = BS * N // NW
CH = 64
NCH = ROWS_PER_W // CH


def _knn_tables_kernel(xall_ref, xrow_ref, w1a_ref, w1d_ref, b1_ref,
                       idx_ref, g_ref, c_ref, work_ref):
    b = pl.program_id(0)
    xall = xall_ref[0]
    xrow = xrow_ref[0]
    inner = lax.dot_general(xrow, xall, (((0,), (0,)), ((), ())),
                            preferred_element_type=jnp.float32)
    xx_all = jnp.sum(xall * xall, axis=0)
    xx_row = jnp.sum(xrow * xrow, axis=0)
    work_ref[...] = 2.0 * inner - xx_row[:, None] - xx_all[None, :]

    col = lax.broadcasted_iota(jnp.int32, (RB, N), 1)
    base = b * N
    for t in range(K):
        cur = work_ref[...]
        m = jnp.max(cur, axis=1, keepdims=True)
        amin = jnp.min(jnp.where(cur == m, col, N), axis=1)
        idx_ref[0, t] = amin + base
        work_ref[...] = jnp.where(col == amin[:, None], -jnp.inf, cur)

    g_ref[...] = lax.dot_general(xrow, w1a_ref[...], (((0,), (0,)), ((), ())),
                                 preferred_element_type=jnp.float32)
    c_ref[...] = lax.dot_general(xrow, w1d_ref[...], (((0,), (0,)), ((), ())),
                                 preferred_element_type=jnp.float32) + b1_ref[...]


def _sc_gather_max_body(idx_hbm, g_hbm, out_hbm, idx_v, buf, acc, sem):
    wid = lax.axis_index("s") * NC + lax.axis_index("c")
    pltpu.sync_copy(idx_hbm.at[wid], idx_v)
    for ci in range(NCH):
        cps = [pltpu.async_copy(g_hbm.at[idx_v.at[t, pl.ds(ci * CH, CH)]],
                                buf.at[pl.ds(t * CH, CH)], sem)
               for t in range(K)]
        for cp in cps:
            cp.wait()

        def body(r, carry):
            for cc in range(EMB // 16):
                v = buf[r, pl.ds(cc * 16, 16)]
                for t in range(1, K):
                    v = jnp.maximum(v, buf[t * CH + r, pl.ds(cc * 16, 16)])
                acc[r, pl.ds(cc * 16, 16)] = v
            return carry

        lax.fori_loop(0, CH, body, 0)
        pltpu.sync_copy(acc,
                        out_hbm.at[pl.ds(wid * ROWS_PER_W + ci * CH, CH)])


def _proj_kernel(m_ref, c_ref, w2_ref, b2_ref, o_ref):
    h = m_ref[...] + c_ref[...]
    h = jnp.where(h >= 0, h, 0.2 * h)
    o = lax.dot_general(w2_ref[...], h, (((0,), (1,)), ((), ())),
                        preferred_element_type=jnp.float32)
    o = o + b2_ref[...]
    o_ref[0] = jnp.where(o >= 0, o, 0.2 * o)


def kernel(xy, W1, b1, W2, b2):
    w1a = W1[:PD]
    w1d = W1[PD:] - W1[:PD]
    b1r = b1[None, :]
    b2r = b2[:, None]

    idx, g, c = pl.pallas_call(
        _knn_tables_kernel,
        grid=(BS, NBLK),
        in_specs=[
            pl.BlockSpec((1, PD, N), lambda b, rb: (b, 0, 0)),
            pl.BlockSpec((1, PD, RB), lambda b, rb: (b, 0, rb)),
            pl.BlockSpec((PD, EMB), lambda b, rb: (0, 0)),
            pl.BlockSpec((PD, EMB), lambda b, rb: (0, 0)),
            pl.BlockSpec((1, EMB), lambda b, rb: (0, 0)),
        ],
        out_specs=[
            pl.BlockSpec((1, K, RB), lambda b, rb: (b * NBLK + rb, 0, 0)),
            pl.BlockSpec((RB, EMB), lambda b, rb: (b * NBLK + rb, 0)),
            pl.BlockSpec((RB, EMB), lambda b, rb: (b * NBLK + rb, 0)),
        ],
        out_shape=[
            jax.ShapeDtypeStruct((NSLAB, K, RB), jnp.int32),
            jax.ShapeDtypeStruct((BS * N, EMB), jnp.float32),
            jax.ShapeDtypeStruct((BS * N, EMB), jnp.float32),
        ],
        scratch_shapes=[pltpu.VMEM((RB, N), jnp.float32)],
    )(xy, xy, w1a, w1d, b1r)

    gather_max = functools.partial(
        pl.kernel,
        mesh=plsc.VectorSubcoreMesh(core_axis_name="c", subcore_axis_name="s"),
        out_type=jax.ShapeDtypeStruct((BS * N, EMB), jnp.float32),
        scratch_types=[
            pltpu.VMEM((K, ROWS_PER_W), jnp.int32),
            pltpu.VMEM((K * CH, EMB), jnp.float32),
            pltpu.VMEM((CH, EMB), jnp.float32),
            pltpu.SemaphoreType.DMA,
        ],
    )(_sc_gather_max_body)
    m = gather_max(idx, g)

    out = pl.pallas_call(
        _proj_kernel,
        grid=(BS, NBLK),
        in_specs=[
            pl.BlockSpec((RB, EMB), lambda b, rb: (b * NBLK + rb, 0)),
            pl.BlockSpec((RB, EMB), lambda b, rb: (b * NBLK + rb, 0)),
            pl.BlockSpec((EMB, EMB), lambda b, rb: (0, 0)),
            pl.BlockSpec((EMB, 1), lambda b, rb: (0, 0)),
        ],
        out_specs=pl.BlockSpec((1, EMB, RB), lambda b, rb: (b, 0, rb)),
        out_shape=jax.ShapeDtypeStruct((BS, EMB, N), jnp.float32),
    )(m, c, W2, b2r)
    return out


# speedup vs baseline: 24.2727x; 1.5067x over previous
import functools

import jax
import jax.numpy as jnp
from jax import lax
from jax.experimental import pallas as pl
from jax.experimental.pallas import tpu as pltpu
from jax.experimental.pallas import tpu_sc as plsc

BS = 8
PD = 2
N = 2048
K = 10
EMB = 128

RB = 512
NBLK = N // RB
NSLAB = BS * NBLK

NC = 2
NS = 16
NW = NC * NS
ROWS_PER_W = BS * N // NW
CH = 64
NCH = ROWS_PER_W // CH


def _knn_tables_kernel(xall_ref, xrow_ref, w1a_ref, w1d_ref, b1_ref,
                       idx_ref, g_ref, c_ref):
    b = pl.program_id(0)
    xall = xall_ref[0]
    xrow = xrow_ref[0]
    inner = lax.dot_general(xrow, xall, (((0,), (0,)), ((), ())),
                            preferred_element_type=jnp.float32)
    xx_all = jnp.sum(xall * xall, axis=0)
    xx_row = jnp.sum(xrow * xrow, axis=0)
    work = 2.0 * inner - xx_row[:, None] - xx_all[None, :]

    NEG = jnp.float32(-jnp.inf)
    sl = [work[:, s * 128:(s + 1) * 128] for s in range(16)]
    C = sl[0]
    for s in range(1, 16):
        C = jnp.maximum(C, sl[s])

    lane = lax.broadcasted_iota(jnp.int32, (RB, 128), 1)
    lane_f = lane.astype(jnp.float32)
    lfs = []
    for t in range(K):
        m = jnp.max(C, axis=1, keepdims=True)
        lf = -jnp.max(jnp.where(C == m, -lane_f, NEG), axis=1, keepdims=True)
        C = jnp.where(lane_f == lf, NEG, C)
        lfs.append(lf)

    TILEa = jnp.zeros((RB, 128), jnp.float32)
    TILEb = jnp.zeros((RB, 128), jnp.float32)
    mod10a = lane % 10
    mod10b = (lane + 8) % 10
    for t in range(K):
        TILEa = jnp.where(mod10a == t, lfs[t], TILEa)
        TILEb = jnp.where(mod10b == t, lfs[t], TILEb)
    IDXa = TILEa.astype(jnp.int32)
    IDXb = TILEb.astype(jnp.int32)
    ga = [jnp.take_along_axis(sl[s], IDXa, axis=1) for s in range(13)]
    gb = [jnp.take_along_axis(sl[s], IDXb, axis=1) for s in range(12, 16)]
    D1 = jnp.full((RB, 128), NEG, jnp.float32)
    for s in range(13):
        msk = (lane // 10) == s if s < 12 else (lane >= 120)
        D1 = jnp.where(msk, ga[s], D1)
    D2 = jnp.full((RB, 128), NEG, jnp.float32)
    for j, s in enumerate(range(12, 16)):
        msk = ((lane + 128) // 10) == s
        D2 = jnp.where(msk, gb[j], D2)
    sua = (lane // 10).astype(jnp.float32) * 128.0
    sub = ((lane + 128) // 10).astype(jnp.float32) * 128.0
    COLa = TILEa + sua
    COLb = TILEb + sub
    NF = jnp.float32(N)

    cmins = []
    for t in range(K):
        m = jnp.maximum(jnp.max(D1, axis=1, keepdims=True),
                        jnp.max(D2, axis=1, keepdims=True))
        c1 = jnp.max(jnp.where(D1 == m, -COLa, NEG), axis=1, keepdims=True)
        c2 = jnp.max(jnp.where(D2 == m, -COLb, NEG), axis=1, keepdims=True)
        cmin = -jnp.maximum(c1, c2)
        D1 = jnp.where(COLa == cmin, NEG, D1)
        D2 = jnp.where(COLb == cmin, NEG, D2)
        cmins.append(cmin)
    base = jnp.float32(b * N)
    idx16 = jnp.concatenate(cmins + [cmins[0]] * 6, axis=1) + base
    idx_ref[0] = idx16.T[:K].astype(jnp.int32)

    g_ref[...] = lax.dot_general(xrow, w1a_ref[...], (((0,), (0,)), ((), ())),
                                 preferred_element_type=jnp.float32)
    c_ref[...] = lax.dot_general(xrow, w1d_ref[...], (((0,), (0,)), ((), ())),
                                 preferred_element_type=jnp.float32) + b1_ref[...]


def _sc_gather_max_body(idx_hbm, g_hbm, out_hbm, idx_v, buf, acc, sem):
    wid = lax.axis_index("s") * NC + lax.axis_index("c")
    pltpu.sync_copy(idx_hbm.at[wid], idx_v)
    for ci in range(NCH):
        cps = [pltpu.async_copy(g_hbm.at[idx_v.at[t, pl.ds(ci * CH, CH)]],
                                buf.at[pl.ds(t * CH, CH)], sem)
               for t in range(K)]
        for cp in cps:
            cp.wait()

        def body(r, carry):
            for cc in range(EMB // 16):
                v = buf[r, pl.ds(cc * 16, 16)]
                for t in range(1, K):
                    v = jnp.maximum(v, buf[t * CH + r, pl.ds(cc * 16, 16)])
                acc[r, pl.ds(cc * 16, 16)] = v
            return carry

        lax.fori_loop(0, CH, body, 0)
        pltpu.sync_copy(acc,
                        out_hbm.at[pl.ds(wid * ROWS_PER_W + ci * CH, CH)])


def _proj_kernel(m_ref, c_ref, w2_ref, b2_ref, o_ref):
    h = m_ref[...] + c_ref[...]
    h = jnp.where(h >= 0, h, 0.2 * h)
    o = lax.dot_general(w2_ref[...], h, (((0,), (1,)), ((), ())),
                        preferred_element_type=jnp.float32)
    o = o + b2_ref[...]
    o_ref[0] = jnp.where(o >= 0, o, 0.2 * o)


def kernel(xy, W1, b1, W2, b2):
    w1a = W1[:PD]
    w1d = W1[PD:] - W1[:PD]
    b1r = b1[None, :]
    b2r = b2[:, None]

    idx, g, c = pl.pallas_call(
        _knn_tables_kernel,
        grid=(BS, NBLK),
        in_specs=[
            pl.BlockSpec((1, PD, N), lambda b, rb: (b, 0, 0)),
            pl.BlockSpec((1, PD, RB), lambda b, rb: (b, 0, rb)),
            pl.BlockSpec((PD, EMB), lambda b, rb: (0, 0)),
            pl.BlockSpec((PD, EMB), lambda b, rb: (0, 0)),
            pl.BlockSpec((1, EMB), lambda b, rb: (0, 0)),
        ],
        out_specs=[
            pl.BlockSpec((1, K, RB), lambda b, rb: (b * NBLK + rb, 0, 0)),
            pl.BlockSpec((RB, EMB), lambda b, rb: (b * NBLK + rb, 0)),
            pl.BlockSpec((RB, EMB), lambda b, rb: (b * NBLK + rb, 0)),
        ],
        out_shape=[
            jax.ShapeDtypeStruct((NSLAB, K, RB), jnp.int32),
            jax.ShapeDtypeStruct((BS * N, EMB), jnp.float32),
            jax.ShapeDtypeStruct((BS * N, EMB), jnp.float32),
        ],
    )(xy, xy, w1a, w1d, b1r)

    gather_max = functools.partial(
        pl.kernel,
        mesh=plsc.VectorSubcoreMesh(core_axis_name="c", subcore_axis_name="s"),
        out_type=jax.ShapeDtypeStruct((BS * N, EMB), jnp.float32),
        scratch_types=[
            pltpu.VMEM((K, ROWS_PER_W), jnp.int32),
            pltpu.VMEM((K * CH, EMB), jnp.float32),
            pltpu.VMEM((CH, EMB), jnp.float32),
            pltpu.SemaphoreType.DMA,
        ],
    )(_sc_gather_max_body)
    m = gather_max(idx, g)

    out = pl.pallas_call(
        _proj_kernel,
        grid=(BS, NBLK),
        in_specs=[
            pl.BlockSpec((RB, EMB), lambda b, rb: (b * NBLK + rb, 0)),
            pl.BlockSpec((RB, EMB), lambda b, rb: (b * NBLK + rb, 0)),
            pl.BlockSpec((EMB, EMB), lambda b, rb: (0, 0)),
            pl.BlockSpec((EMB, 1), lambda b, rb: (0, 0)),
        ],
        out_specs=pl.BlockSpec((1, EMB, RB), lambda b, rb: (b, 0, rb)),
        out_shape=jax.ShapeDtypeStruct((BS, EMB, N), jnp.float32),
    )(m, c, W2, b2r)
    return out
